# baseline (speedup 1.0000x reference)
import jax
import jax.numpy as jnp
from jax import lax
from jax.experimental import pallas as pl
from jax.experimental.pallas import tpu as pltpu
from jax.experimental.pallas import tpu_sc as plsc

N = 10000
E = 160000
IN_DIM = 256
HID_DIM = 512
OUT_DIM = 256

NC = 2
NS = 16
B = 128
NB = 80
E_PAD = NS * NB * B
RPT = 640
N_PAD = NS * RPT
C = 64
K = 4
KB = K * B
NG = NB // K
NSLOT = 2
DEPTH = 1

R = 1024
G = 10

NCH1 = IN_DIM // C
NCH2 = HID_DIM // C


def _make_sc_agg(n_chunks, with_cnt):
  mesh = plsc.VectorSubcoreMesh(core_axis_name="c", subcore_axis_name="s")
  out_type = [jax.ShapeDtypeStruct((N_PAD, C), jnp.float32)
              for _ in range(n_chunks)]
  if with_cnt:
    out_type.append(jax.ShapeDtypeStruct((N_PAD,), jnp.float32))
  scratch = [
      pltpu.VMEM((NG, KB), jnp.int32),
      pltpu.VMEM((NG, KB), jnp.int32),
      pltpu.VMEM((NSLOT, KB, C), jnp.float32),
      pltpu.VMEM_SHARED((N_PAD, C), jnp.float32),
      pltpu.SemaphoreType.DMA,
      pltpu.SemaphoreType.DMA,
  ]
  if with_cnt:
    scratch += [
        pltpu.VMEM((KB,), jnp.float32),
        pltpu.VMEM((RPT,), jnp.float32),
        pltpu.VMEM_SHARED((N_PAD,), jnp.float32),
        pltpu.SemaphoreType.DMA,
    ]

  def body(*args):
    a = list(args)
    src_hbm, dst_hbm, z2d_hbm = a[:3]
    a = a[3:]
    if with_cnt:
      ones_hbm, z1d_hbm = a[:2]
      a = a[2:]
    tables = a[:n_chunks]
    a = a[n_chunks:]
    outs = a[:n_chunks]
    a = a[n_chunks:]
    if with_cnt:
      cnt_out = a[0]
      a = a[1:]
    src_v, dst_v, rowsr_v, acc_sh, sem_g, sem_s = a[:6]
    if with_cnt:
      ones_v, z1d_v, cnt_sh, sem_c = a[6:10]

    c = lax.axis_index("c")
    s = lax.axis_index("s")
    row0 = s * RPT

    pltpu.sync_copy(src_hbm.at[s], src_v)
    pltpu.sync_copy(dst_hbm.at[s], dst_v)
    if with_cnt:
      pltpu.sync_copy(ones_hbm, ones_v)
      pltpu.sync_copy(z1d_hbm, z1d_v)

    for k in range(n_chunks):
      @pl.when(c == (k % NC))
      def _chunk(k=k):
        pltpu.sync_copy(z2d_hbm, rowsr_v.at[0].at[pl.ds(0, 128)])
        for j in range(RPT // 128):
          pltpu.sync_copy(rowsr_v.at[0].at[pl.ds(0, 128)],
                          acc_sh.at[pl.ds(row0 + j * 128, 128)])
        if with_cnt and k == 0:
          pltpu.sync_copy(z1d_v, cnt_sh.at[pl.ds(row0, RPT)])
        plsc.subcore_barrier()

        for p in range(DEPTH):
          pltpu.async_copy(tables[k].at[src_v.at[p]],
                           rowsr_v.at[p], sem_g)

        def step(g, carry, k=k):
          @pl.when(g >= DEPTH)
          def _drain():
            pltpu.make_async_copy(
                rowsr_v.at[lax.rem(g - DEPTH, NSLOT)],
                acc_sh.at[dst_v.at[g - DEPTH]], sem_s).wait()

          @pl.when(g + DEPTH < NG)
          def _prefetch():
            pltpu.async_copy(tables[k].at[src_v.at[g + DEPTH]],
                             rowsr_v.at[lax.rem(g + DEPTH, NSLOT)], sem_g)

          pltpu.make_async_copy(tables[k].at[src_v.at[g]],
                                rowsr_v.at[lax.rem(g, NSLOT)], sem_g).wait()
          pltpu.async_copy(rowsr_v.at[lax.rem(g, NSLOT)],
                           acc_sh.at[dst_v.at[g]], sem_s,
                           add=True)
          if with_cnt and k == 0:
            pltpu.async_copy(ones_v, cnt_sh.at[dst_v.at[g]],
                             sem_c, add=True)
          return carry

        lax.fori_loop(0, NG, step, 0)
        for g in range(max(0, NG - DEPTH), NG):
          pltpu.make_async_copy(rowsr_v.at[g % NSLOT],
                                acc_sh.at[dst_v.at[g]],
                                sem_s).wait()
        if with_cnt and k == 0:
          def drain_cnt(g, carry):
            pltpu.make_async_copy(ones_v, cnt_sh.at[dst_v.at[g]],
                                  sem_c).wait()
            return carry
          lax.fori_loop(0, NG, drain_cnt, 0)
        plsc.subcore_barrier()

        for j in range(RPT // 128):
          pltpu.sync_copy(acc_sh.at[pl.ds(row0 + j * 128, 128)],
                          rowsr_v.at[0].at[pl.ds(0, 128)])
          pltpu.sync_copy(rowsr_v.at[0].at[pl.ds(0, 128)],
                          outs[k].at[pl.ds(row0 + j * 128, 128)])
        if with_cnt and k == 0:
          pltpu.sync_copy(cnt_sh.at[pl.ds(row0, RPT)], z1d_v)
          pltpu.sync_copy(z1d_v, cnt_out.at[pl.ds(row0, RPT)])

    return None

  return pl.kernel(
      body, out_type=out_type, mesh=mesh, scratch_types=scratch,
      compiler_params=pltpu.CompilerParams(use_tc_tiling_on_sc=False))


_sc_agg_l1 = _make_sc_agg(NCH1, with_cnt=True)
_sc_agg_l2 = _make_sc_agg(NCH2, with_cnt=False)


def _tc_layer1(aggs, cnt, x, wl, bl, wr):
  def bodyfn(*refs):
    a_refs = refs[:NCH1]
    cnt_ref, x_ref, wl_ref, bl_ref, wr_ref = refs[NCH1:NCH1 + 5]
    h_refs = refs[NCH1 + 5:]
    inv = 1.0 / jnp.maximum(cnt_ref[...], 1.0)
    agg = jnp.concatenate([r[...] for r in a_refs], axis=1) * inv
    h = jnp.dot(agg, wl_ref[...], preferred_element_type=jnp.float32)
    h = h + jnp.dot(x_ref[...], wr_ref[...], preferred_element_type=jnp.float32)
    h = jnp.maximum(h + bl_ref[...], 0.0)
    for k, hr in enumerate(h_refs):
      hr[...] = h[:, k * C:(k + 1) * C]

  return pl.pallas_call(
      bodyfn,
      grid=(G,),
      in_specs=[pl.BlockSpec((R, C), lambda i: (i, 0))] * NCH1 + [
          pl.BlockSpec((R, 1), lambda i: (i, 0)),
          pl.BlockSpec((R, IN_DIM), lambda i: (i, 0)),
          pl.BlockSpec((IN_DIM, HID_DIM), lambda i: (0, 0)),
          pl.BlockSpec((1, HID_DIM), lambda i: (0, 0)),
          pl.BlockSpec((IN_DIM, HID_DIM), lambda i: (0, 0)),
      ],
      out_specs=[pl.BlockSpec((R, C), lambda i: (i, 0))] * NCH2,
      out_shape=[jax.ShapeDtypeStruct((N_PAD, C), jnp.float32)] * NCH2,
  )(*aggs, cnt, x, wl, bl, wr)


def _tc_layer2(aggs, cnt, hs, wl, bl, wr):
  def bodyfn(*refs):
    a_refs = refs[:NCH2]
    cnt_ref = refs[NCH2]
    h_refs = refs[NCH2 + 1:2 * NCH2 + 1]
    wl_ref, bl_ref, wr_ref = refs[2 * NCH2 + 1:2 * NCH2 + 4]
    out_ref = refs[-1]
    inv = 1.0 / jnp.maximum(cnt_ref[...], 1.0)
    agg = jnp.concatenate([r[...] for r in a_refs], axis=1) * inv
    h = jnp.concatenate([r[...] for r in h_refs], axis=1)
    o = jnp.dot(agg, wl_ref[...], preferred_element_type=jnp.float32)
    o = o + jnp.dot(h, wr_ref[...], preferred_element_type=jnp.float32)
    out_ref[...] = o + bl_ref[...]

  return pl.pallas_call(
      bodyfn,
      grid=(G,),
      in_specs=[pl.BlockSpec((R, C), lambda i: (i, 0))] * NCH2 + [
          pl.BlockSpec((R, 1), lambda i: (i, 0)),
      ] + [pl.BlockSpec((R, C), lambda i: (i, 0))] * NCH2 + [
          pl.BlockSpec((HID_DIM, OUT_DIM), lambda i: (0, 0)),
          pl.BlockSpec((1, OUT_DIM), lambda i: (0, 0)),
          pl.BlockSpec((HID_DIM, OUT_DIM), lambda i: (0, 0)),
      ],
      out_specs=pl.BlockSpec((R, OUT_DIM), lambda i: (i, 0)),
      out_shape=jax.ShapeDtypeStruct((N, OUT_DIM), jnp.float32),
  )(*aggs, cnt, *hs, wl, bl, wr)


def kernel(x, edge_index, Wl1, bl1, Wr1, Wl2, bl2, Wr2):
  ei = edge_index.astype(jnp.int32)
  src = jnp.concatenate(
      [ei[0], jnp.zeros((E_PAD - E,), jnp.int32)]).reshape(NS, NG, KB)
  dst = jnp.concatenate(
      [ei[1], jnp.full((E_PAD - E,), N, jnp.int32)]).reshape(NS, NG, KB)
  z2d = jnp.zeros((128, C), jnp.float32)
  ones1 = jnp.ones((KB,), jnp.float32)
  z1d = jnp.zeros((RPT,), jnp.float32)

  xc = [x[:, k * C:(k + 1) * C] for k in range(NCH1)]
  *a, cnt = _sc_agg_l1(src, dst, z2d, ones1, z1d, *xc)
  cnt2 = cnt.reshape(N_PAD, 1)

  h = _tc_layer1(a, cnt2, x, Wl1.T, bl1.reshape(1, -1), Wr1.T)

  g = _sc_agg_l2(src, dst, z2d, *h)

  return _tc_layer2(g, cnt2, h, Wl2.T, bl2.reshape(1, -1), Wr2.T)


# speedup vs baseline: 1.0351x; 1.0351x over previous
import jax
import jax.numpy as jnp
from jax import lax
from jax.experimental import pallas as pl
from jax.experimental.pallas import tpu as pltpu
from jax.experimental.pallas import tpu_sc as plsc

N = 10000
E = 160000
IN_DIM = 256
HID_DIM = 512
OUT_DIM = 256

NC = 2
NS = 16
B = 128
NB = 80
E_PAD = NS * NB * B
RPT = 640
N_PAD = NS * RPT
C = 64
K = 1
KB = K * B
NG = NB // K
NSLOT = 6
DEPTH = 3

R = 1024
G = 10

NCH1 = IN_DIM // C
NCH2 = HID_DIM // C


def _make_sc_agg(n_chunks, with_cnt):
  mesh = plsc.VectorSubcoreMesh(core_axis_name="c", subcore_axis_name="s")
  out_type = [jax.ShapeDtypeStruct((N_PAD, C), jnp.float32)
              for _ in range(n_chunks)]
  if with_cnt:
    out_type.append(jax.ShapeDtypeStruct((N_PAD,), jnp.float32))
  scratch = [
      pltpu.VMEM((NG, KB), jnp.int32),
      pltpu.VMEM((NG, KB), jnp.int32),
      pltpu.VMEM((NSLOT, KB, C), jnp.float32),
      pltpu.VMEM_SHARED((N_PAD, C), jnp.float32),
      pltpu.SemaphoreType.DMA,
      pltpu.SemaphoreType.DMA,
  ]
  if with_cnt:
    scratch += [
        pltpu.VMEM((KB,), jnp.float32),
        pltpu.VMEM((RPT,), jnp.float32),
        pltpu.VMEM_SHARED((N_PAD,), jnp.float32),
        pltpu.SemaphoreType.DMA,
    ]

  def body(*args):
    a = list(args)
    src_hbm, dst_hbm, z2d_hbm = a[:3]
    a = a[3:]
    if with_cnt:
      ones_hbm, z1d_hbm = a[:2]
      a = a[2:]
    tables = a[:n_chunks]
    a = a[n_chunks:]
    outs = a[:n_chunks]
    a = a[n_chunks:]
    if with_cnt:
      cnt_out = a[0]
      a = a[1:]
    src_v, dst_v, rowsr_v, acc_sh, sem_g, sem_s = a[:6]
    if with_cnt:
      ones_v, z1d_v, cnt_sh, sem_c = a[6:10]

    c = lax.axis_index("c")
    s = lax.axis_index("s")
    row0 = s * RPT

    pltpu.sync_copy(src_hbm.at[s], src_v)
    pltpu.sync_copy(dst_hbm.at[s], dst_v)
    if with_cnt:
      pltpu.sync_copy(ones_hbm, ones_v)
      pltpu.sync_copy(z1d_hbm, z1d_v)

    for k in range(n_chunks):
      @pl.when(c == (k % NC))
      def _chunk(k=k):
        pltpu.sync_copy(z2d_hbm, rowsr_v.at[0].at[pl.ds(0, 128)])
        for j in range(RPT // 128):
          pltpu.sync_copy(rowsr_v.at[0].at[pl.ds(0, 128)],
                          acc_sh.at[pl.ds(row0 + j * 128, 128)])
        if with_cnt and k == 0:
          pltpu.sync_copy(z1d_v, cnt_sh.at[pl.ds(row0, RPT)])
        plsc.subcore_barrier()

        for p in range(DEPTH):
          pltpu.async_copy(tables[k].at[src_v.at[p]],
                           rowsr_v.at[p], sem_g)

        def step(g, carry, k=k):
          @pl.when(g >= DEPTH)
          def _drain():
            pltpu.make_async_copy(
                rowsr_v.at[lax.rem(g - DEPTH, NSLOT)],
                acc_sh.at[dst_v.at[g - DEPTH]], sem_s).wait()

          @pl.when(g + DEPTH < NG)
          def _prefetch():
            pltpu.async_copy(tables[k].at[src_v.at[g + DEPTH]],
                             rowsr_v.at[lax.rem(g + DEPTH, NSLOT)], sem_g)

          pltpu.make_async_copy(tables[k].at[src_v.at[g]],
                                rowsr_v.at[lax.rem(g, NSLOT)], sem_g).wait()
          pltpu.async_copy(rowsr_v.at[lax.rem(g, NSLOT)],
                           acc_sh.at[dst_v.at[g]], sem_s,
                           add=True)
          if with_cnt and k == 0:
            pltpu.async_copy(ones_v, cnt_sh.at[dst_v.at[g]],
                             sem_c, add=True)
          return carry

        lax.fori_loop(0, NG, step, 0)
        for g in range(max(0, NG - DEPTH), NG):
          pltpu.make_async_copy(rowsr_v.at[g % NSLOT],
                                acc_sh.at[dst_v.at[g]],
                                sem_s).wait()
        if with_cnt and k == 0:
          def drain_cnt(g, carry):
            pltpu.make_async_copy(ones_v, cnt_sh.at[dst_v.at[g]],
                                  sem_c).wait()
            return carry
          lax.fori_loop(0, NG, drain_cnt, 0)
        plsc.subcore_barrier()

        for j in range(RPT // 128):
          pltpu.sync_copy(acc_sh.at[pl.ds(row0 + j * 128, 128)],
                          rowsr_v.at[0].at[pl.ds(0, 128)])
          pltpu.sync_copy(rowsr_v.at[0].at[pl.ds(0, 128)],
                          outs[k].at[pl.ds(row0 + j * 128, 128)])
        if with_cnt and k == 0:
          pltpu.sync_copy(cnt_sh.at[pl.ds(row0, RPT)], z1d_v)
          pltpu.sync_copy(z1d_v, cnt_out.at[pl.ds(row0, RPT)])

    return None

  return pl.kernel(
      body, out_type=out_type, mesh=mesh, scratch_types=scratch,
      compiler_params=pltpu.CompilerParams(use_tc_tiling_on_sc=False))


_sc_agg_l1 = _make_sc_agg(NCH1, with_cnt=True)
_sc_agg_l2 = _make_sc_agg(NCH2, with_cnt=False)


def _tc_root(x, w, d_in, d_out, n_rows):
  def bodyfn(x_ref, w_ref, o_ref):
    o_ref[...] = jnp.dot(x_ref[...], w_ref[...],
                         preferred_element_type=jnp.float32)

  return pl.pallas_call(
      bodyfn,
      grid=(G,),
      in_specs=[
          pl.BlockSpec((R, d_in), lambda i: (i, 0)),
          pl.BlockSpec((d_in, d_out), lambda i: (0, 0)),
      ],
      out_specs=pl.BlockSpec((R, d_out), lambda i: (i, 0)),
      out_shape=jax.ShapeDtypeStruct((n_rows, d_out), jnp.float32),
  )(x, w)


def _tc_root_chunked(hs, w, d_in, d_out):
  def bodyfn(*refs):
    h_refs = refs[:NCH2]
    w_ref, o_ref = refs[NCH2], refs[NCH2 + 1]
    h = jnp.concatenate([r[...] for r in h_refs], axis=1)
    o_ref[...] = jnp.dot(h, w_ref[...], preferred_element_type=jnp.float32)

  return pl.pallas_call(
      bodyfn,
      grid=(G,),
      in_specs=[pl.BlockSpec((R, C), lambda i: (i, 0))] * NCH2 + [
          pl.BlockSpec((d_in, d_out), lambda i: (0, 0)),
      ],
      out_specs=pl.BlockSpec((R, d_out), lambda i: (i, 0)),
      out_shape=jax.ShapeDtypeStruct((N_PAD, d_out), jnp.float32),
  )(*hs, w)


def _tc_comb1(aggs, cnt, xr, wl, bl):
  def bodyfn(*refs):
    a_refs = refs[:NCH1]
    cnt_ref, xr_ref, wl_ref, bl_ref = refs[NCH1:NCH1 + 4]
    h_refs = refs[NCH1 + 4:]
    inv = 1.0 / jnp.maximum(cnt_ref[...], 1.0)
    agg = jnp.concatenate([r[...] for r in a_refs], axis=1) * inv
    h = jnp.dot(agg, wl_ref[...], preferred_element_type=jnp.float32)
    h = jnp.maximum(h + xr_ref[...] + bl_ref[...], 0.0)
    for k, hr in enumerate(h_refs):
      hr[...] = h[:, k * C:(k + 1) * C]

  return pl.pallas_call(
      bodyfn,
      grid=(G,),
      in_specs=[pl.BlockSpec((R, C), lambda i: (i, 0))] * NCH1 + [
          pl.BlockSpec((R, 1), lambda i: (i, 0)),
          pl.BlockSpec((R, HID_DIM), lambda i: (i, 0)),
          pl.BlockSpec((IN_DIM, HID_DIM), lambda i: (0, 0)),
          pl.BlockSpec((1, HID_DIM), lambda i: (0, 0)),
      ],
      out_specs=[pl.BlockSpec((R, C), lambda i: (i, 0))] * NCH2,
      out_shape=[jax.ShapeDtypeStruct((N_PAD, C), jnp.float32)] * NCH2,
  )(*aggs, cnt, xr, wl, bl)


def _tc_comb2(aggs, cnt, hr, wl, bl):
  def bodyfn(*refs):
    a_refs = refs[:NCH2]
    cnt_ref, hr_ref, wl_ref, bl_ref, out_ref = refs[NCH2:NCH2 + 5]
    inv = 1.0 / jnp.maximum(cnt_ref[...], 1.0)
    agg = jnp.concatenate([r[...] for r in a_refs], axis=1) * inv
    o = jnp.dot(agg, wl_ref[...], preferred_element_type=jnp.float32)
    out_ref[...] = o + hr_ref[...] + bl_ref[...]

  return pl.pallas_call(
      bodyfn,
      grid=(G,),
      in_specs=[pl.BlockSpec((R, C), lambda i: (i, 0))] * NCH2 + [
          pl.BlockSpec((R, 1), lambda i: (i, 0)),
          pl.BlockSpec((R, OUT_DIM), lambda i: (i, 0)),
          pl.BlockSpec((HID_DIM, OUT_DIM), lambda i: (0, 0)),
          pl.BlockSpec((1, OUT_DIM), lambda i: (0, 0)),
      ],
      out_specs=pl.BlockSpec((R, OUT_DIM), lambda i: (i, 0)),
      out_shape=jax.ShapeDtypeStruct((N, OUT_DIM), jnp.float32),
  )(*aggs, cnt, hr, wl, bl)


def kernel(x, edge_index, Wl1, bl1, Wr1, Wl2, bl2, Wr2):
  ei = edge_index.astype(jnp.int32)
  src = jnp.concatenate(
      [ei[0], jnp.zeros((E_PAD - E,), jnp.int32)]).reshape(NS, NG, KB)
  dst = jnp.concatenate(
      [ei[1], jnp.full((E_PAD - E,), N, jnp.int32)]).reshape(NS, NG, KB)
  z2d = jnp.zeros((128, C), jnp.float32)
  ones1 = jnp.ones((KB,), jnp.float32)
  z1d = jnp.zeros((RPT,), jnp.float32)

  xc = [x[:, k * C:(k + 1) * C] for k in range(NCH1)]
  *a, cnt = _sc_agg_l1(src, dst, z2d, ones1, z1d, *xc)
  cnt2 = cnt.reshape(N_PAD, 1)

  xr = _tc_root(x, Wr1.T, IN_DIM, HID_DIM, N_PAD)
  h = _tc_comb1(a, cnt2, xr, Wl1.T, bl1.reshape(1, -1))

  g = _sc_agg_l2(src, dst, z2d, *h)

  hr = _tc_root_chunked(h, Wr2.T, HID_DIM, OUT_DIM)
  return _tc_comb2(g, cnt2, hr, Wl2.T, bl2.reshape(1, -1))


# speedup vs baseline: 2.2181x; 2.1429x over previous
import jax
import jax.numpy as jnp
from jax import lax
from jax.experimental import pallas as pl
from jax.experimental.pallas import tpu as pltpu
from jax.experimental.pallas import tpu_sc as plsc

N = 10000
E = 160000
IN_DIM = 256
HID_DIM = 512
OUT_DIM = 256

NC = 2
NS = 16
B = 128
NB = 80
E_PAD = NS * NB * B
RPT = 640
N_PAD = NS * RPT
C = 64
K = 1
KB = K * B
NG = NB // K
NSLOT = 6
DEPTH = 3

R = 1024
G = 10

NCH1 = IN_DIM // C
NCH2 = HID_DIM // C


def _make_sc_agg(n_chunks, with_cnt):
  mesh = plsc.VectorSubcoreMesh(core_axis_name="c", subcore_axis_name="s")
  out_type = [jax.ShapeDtypeStruct((N_PAD, C), jnp.float32)
              for _ in range(n_chunks)]
  if with_cnt:
    out_type.append(jax.ShapeDtypeStruct((N_PAD,), jnp.float32))
  scratch = [
      pltpu.VMEM((NG, KB), jnp.int32),
      pltpu.VMEM((NG, KB), jnp.int32),
      pltpu.VMEM((NSLOT, KB, C), jnp.float32),
      pltpu.VMEM_SHARED((N_PAD, C), jnp.float32),
      pltpu.SemaphoreType.DMA,
      pltpu.SemaphoreType.DMA,
  ]
  if with_cnt:
    scratch += [
        pltpu.VMEM((KB,), jnp.float32),
        pltpu.VMEM((RPT,), jnp.float32),
        pltpu.VMEM_SHARED((N_PAD,), jnp.float32),
        pltpu.SemaphoreType.DMA,
    ]

  def body(*args):
    a = list(args)
    src_hbm, dst_hbm, z2d_hbm = a[:3]
    a = a[3:]
    if with_cnt:
      ones_hbm, z1d_hbm = a[:2]
      a = a[2:]
    tables = a[:n_chunks]
    a = a[n_chunks:]
    outs = a[:n_chunks]
    a = a[n_chunks:]
    if with_cnt:
      cnt_out = a[0]
      a = a[1:]
    src_v, dst_v, rowsr_v, acc_sh, sem_g, sem_s = a[:6]
    if with_cnt:
      ones_v, z1d_v, cnt_sh, sem_c = a[6:10]

    c = lax.axis_index("c")
    s = lax.axis_index("s")
    row0 = s * RPT

    pltpu.sync_copy(src_hbm.at[s], src_v)
    pltpu.sync_copy(dst_hbm.at[s], dst_v)
    if with_cnt:
      pltpu.sync_copy(ones_hbm, ones_v)
      pltpu.sync_copy(z1d_hbm, z1d_v)

    for k in range(n_chunks):
      @pl.when(c == (k % NC))
      def _chunk(k=k):
        pltpu.sync_copy(z2d_hbm, rowsr_v.at[0].at[pl.ds(0, 128)])
        for j in range(RPT // 128):
          pltpu.sync_copy(rowsr_v.at[0].at[pl.ds(0, 128)],
                          acc_sh.at[pl.ds(row0 + j * 128, 128)])
        if with_cnt and k == 0:
          pltpu.sync_copy(z1d_v, cnt_sh.at[pl.ds(row0, RPT)])
        plsc.subcore_barrier()

        for p in range(DEPTH):
          pltpu.async_copy(tables[k].at[src_v.at[p]],
                           rowsr_v.at[p], sem_g)

        def step(g, carry, k=k):
          @pl.when(g >= DEPTH)
          def _drain():
            pltpu.make_async_copy(
                rowsr_v.at[lax.rem(g - DEPTH, NSLOT)],
                acc_sh.at[dst_v.at[g - DEPTH]], sem_s).wait()

          @pl.when(g + DEPTH < NG)
          def _prefetch():
            pltpu.async_copy(tables[k].at[src_v.at[g + DEPTH]],
                             rowsr_v.at[lax.rem(g + DEPTH, NSLOT)], sem_g)

          pltpu.make_async_copy(tables[k].at[src_v.at[g]],
                                rowsr_v.at[lax.rem(g, NSLOT)], sem_g).wait()
          pltpu.async_copy(rowsr_v.at[lax.rem(g, NSLOT)],
                           acc_sh.at[dst_v.at[g]], sem_s,
                           add=True)
          if with_cnt and k == 0:
            pltpu.async_copy(ones_v, cnt_sh.at[dst_v.at[g]],
                             sem_c, add=True)
          return carry

        lax.fori_loop(0, NG, step, 0)
        for g in range(max(0, NG - DEPTH), NG):
          pltpu.make_async_copy(rowsr_v.at[g % NSLOT],
                                acc_sh.at[dst_v.at[g]],
                                sem_s).wait()
        if with_cnt and k == 0:
          def drain_cnt(g, carry):
            pltpu.make_async_copy(ones_v, cnt_sh.at[dst_v.at[g]],
                                  sem_c).wait()
            return carry
          lax.fori_loop(0, NG, drain_cnt, 0)
        plsc.subcore_barrier()

        for j in range(RPT // 128):
          pltpu.sync_copy(acc_sh.at[pl.ds(row0 + j * 128, 128)],
                          rowsr_v.at[0].at[pl.ds(0, 128)])
          pltpu.sync_copy(rowsr_v.at[0].at[pl.ds(0, 128)],
                          outs[k].at[pl.ds(row0 + j * 128, 128)])
        if with_cnt and k == 0:
          pltpu.sync_copy(cnt_sh.at[pl.ds(row0, RPT)], z1d_v)
          pltpu.sync_copy(z1d_v, cnt_out.at[pl.ds(row0, RPT)])

    return None

  return pl.kernel(
      body, out_type=out_type, mesh=mesh, scratch_types=scratch,
      compiler_params=pltpu.CompilerParams(use_tc_tiling_on_sc=False))


_sc_agg_l1 = _make_sc_agg(NCH1, with_cnt=True)
_sc_agg_l2 = _make_sc_agg(NCH2, with_cnt=False)


def _tc_root(x, w, d_in, d_out, n_rows):
  def bodyfn(x_ref, w_ref, o_ref):
    o_ref[...] = jnp.dot(x_ref[...], w_ref[...],
                         preferred_element_type=jnp.float32)

  return pl.pallas_call(
      bodyfn,
      grid=(G,),
      in_specs=[
          pl.BlockSpec((R, d_in), lambda i: (i, 0)),
          pl.BlockSpec((d_in, d_out), lambda i: (0, 0)),
      ],
      out_specs=pl.BlockSpec((R, d_out), lambda i: (i, 0)),
      out_shape=jax.ShapeDtypeStruct((n_rows, d_out), jnp.float32),
  )(x, w)


def _tc_root_chunked(hs, w, d_in, d_out):
  def bodyfn(*refs):
    h_refs = refs[:NCH2]
    w_ref, o_ref = refs[NCH2], refs[NCH2 + 1]
    h = jnp.concatenate([r[...] for r in h_refs], axis=1)
    o_ref[...] = jnp.dot(h, w_ref[...], preferred_element_type=jnp.float32)

  return pl.pallas_call(
      bodyfn,
      grid=(G,),
      in_specs=[pl.BlockSpec((R, C), lambda i: (i, 0))] * NCH2 + [
          pl.BlockSpec((d_in, d_out), lambda i: (0, 0)),
      ],
      out_specs=pl.BlockSpec((R, d_out), lambda i: (i, 0)),
      out_shape=jax.ShapeDtypeStruct((N_PAD, d_out), jnp.float32),
  )(*hs, w)


def _tc_comb1(aggs, cnt, xr, wl, bl):
  def bodyfn(*refs):
    a_refs = refs[:NCH1]
    cnt_ref, xr_ref, wl_ref, bl_ref = refs[NCH1:NCH1 + 4]
    h_refs = refs[NCH1 + 4:]
    inv = 1.0 / jnp.maximum(cnt_ref[...], 1.0)
    agg = jnp.concatenate([r[...] for r in a_refs], axis=1) * inv
    h = jnp.dot(agg, wl_ref[...], preferred_element_type=jnp.float32)
    h = jnp.maximum(h + xr_ref[...] + bl_ref[...], 0.0)
    for k, hr in enumerate(h_refs):
      hr[...] = h[:, k * C:(k + 1) * C]

  return pl.pallas_call(
      bodyfn,
      grid=(G,),
      in_specs=[pl.BlockSpec((R, C), lambda i: (i, 0))] * NCH1 + [
          pl.BlockSpec((R, 1), lambda i: (i, 0)),
          pl.BlockSpec((R, HID_DIM), lambda i: (i, 0)),
          pl.BlockSpec((IN_DIM, HID_DIM), lambda i: (0, 0)),
          pl.BlockSpec((1, HID_DIM), lambda i: (0, 0)),
      ],
      out_specs=[pl.BlockSpec((R, C), lambda i: (i, 0))] * NCH2,
      out_shape=[jax.ShapeDtypeStruct((N_PAD, C), jnp.float32)] * NCH2,
  )(*aggs, cnt, xr, wl, bl)


def _tc_comb2(aggs, cnt, hr, wl, bl):
  def bodyfn(*refs):
    a_refs = refs[:NCH2]
    cnt_ref, hr_ref, wl_ref, bl_ref, out_ref = refs[NCH2:NCH2 + 5]
    inv = 1.0 / jnp.maximum(cnt_ref[...], 1.0)
    agg = jnp.concatenate([r[...] for r in a_refs], axis=1) * inv
    o = jnp.dot(agg, wl_ref[...], preferred_element_type=jnp.float32)
    out_ref[...] = o + hr_ref[...] + bl_ref[...]

  return pl.pallas_call(
      bodyfn,
      grid=(G,),
      in_specs=[pl.BlockSpec((R, C), lambda i: (i, 0))] * NCH2 + [
          pl.BlockSpec((R, 1), lambda i: (i, 0)),
          pl.BlockSpec((R, OUT_DIM), lambda i: (i, 0)),
          pl.BlockSpec((HID_DIM, OUT_DIM), lambda i: (0, 0)),
          pl.BlockSpec((1, OUT_DIM), lambda i: (0, 0)),
      ],
      out_specs=pl.BlockSpec((R, OUT_DIM), lambda i: (i, 0)),
      out_shape=jax.ShapeDtypeStruct((N, OUT_DIM), jnp.float32),
  )(*aggs, cnt, hr, wl, bl)


def kernel(x, edge_index, Wl1, bl1, Wr1, Wl2, bl2, Wr2):
  ei = edge_index.astype(jnp.int32)
  npad = E_PAD - E
  pad_iota = jnp.arange(npad, dtype=jnp.int32)
  src = jnp.concatenate(
      [ei[0], pad_iota % N]).reshape(NS, NG, KB)
  dst = jnp.concatenate(
      [ei[1], N + pad_iota % (N_PAD - N)]).reshape(NS, NG, KB)
  z2d = jnp.zeros((128, C), jnp.float32)
  ones1 = jnp.ones((KB,), jnp.float32)
  z1d = jnp.zeros((RPT,), jnp.float32)

  xc = [x[:, k * C:(k + 1) * C] for k in range(NCH1)]
  *a, cnt = _sc_agg_l1(src, dst, z2d, ones1, z1d, *xc)
  cnt2 = cnt.reshape(N_PAD, 1)

  xr = _tc_root(x, Wr1.T, IN_DIM, HID_DIM, N_PAD)
  h = _tc_comb1(a, cnt2, xr, Wl1.T, bl1.reshape(1, -1))

  g = _sc_agg_l2(src, dst, z2d, *h)

  hr = _tc_root_chunked(h, Wr2.T, HID_DIM, OUT_DIM)
  return _tc_comb2(g, cnt2, hr, Wl2.T, bl2.reshape(1, -1))


# speedup vs baseline: 2.2497x; 1.0142x over previous
import jax
import jax.numpy as jnp
from jax import lax
from jax.experimental import pallas as pl
from jax.experimental.pallas import tpu as pltpu
from jax.experimental.pallas import tpu_sc as plsc

N = 10000
E = 160000
IN_DIM = 256
HID_DIM = 512
OUT_DIM = 256

NC = 2
NS = 16
B = 128
NB = 80
E_PAD = NS * NB * B
RPT = 640
N_PAD = NS * RPT
C = 64
K = 1
KB = K * B
NG = NB // K
NSLOT = 6
DEPTH = 3

R = 1024
G = 10

NCH1 = IN_DIM // C
NCH2 = HID_DIM // C


def _make_sc_agg(n_chunks, with_cnt):
  mesh = plsc.VectorSubcoreMesh(core_axis_name="c", subcore_axis_name="s")
  out_type = [jax.ShapeDtypeStruct((N_PAD, C), jnp.float32)
              for _ in range(n_chunks)]
  if with_cnt:
    out_type.append(jax.ShapeDtypeStruct((N_PAD,), jnp.float32))
  scratch = [
      pltpu.VMEM((NG, KB), jnp.int32),
      pltpu.VMEM((NG, KB), jnp.int32),
      pltpu.VMEM((NSLOT, KB, C), jnp.float32),
      pltpu.VMEM_SHARED((N_PAD, C), jnp.float32),
      pltpu.SemaphoreType.DMA,
      pltpu.SemaphoreType.DMA,
  ]
  if with_cnt:
    scratch += [
        pltpu.VMEM((KB,), jnp.float32),
        pltpu.VMEM((RPT,), jnp.float32),
        pltpu.VMEM_SHARED((N_PAD,), jnp.float32),
        pltpu.SemaphoreType.DMA,
    ]

  def body(*args):
    a = list(args)
    src_hbm, dst_hbm, z2d_hbm = a[:3]
    a = a[3:]
    if with_cnt:
      ones_hbm, z1d_hbm = a[:2]
      a = a[2:]
    tables = a[:n_chunks]
    a = a[n_chunks:]
    outs = a[:n_chunks]
    a = a[n_chunks:]
    if with_cnt:
      cnt_out = a[0]
      a = a[1:]
    src_v, dst_v, rowsr_v, acc_sh, sem_g, sem_s = a[:6]
    if with_cnt:
      ones_v, z1d_v, cnt_sh, sem_c = a[6:10]

    c = lax.axis_index("c")
    s = lax.axis_index("s")
    row0 = s * RPT

    pltpu.sync_copy(src_hbm.at[s], src_v)
    pltpu.sync_copy(dst_hbm.at[s], dst_v)
    if with_cnt:
      pltpu.sync_copy(ones_hbm, ones_v)
      pltpu.sync_copy(z1d_hbm, z1d_v)

    for k in range(n_chunks):
      @pl.when(c == (k % NC))
      def _chunk(k=k):
        pltpu.sync_copy(z2d_hbm, rowsr_v.at[0].at[pl.ds(0, 128)])
        for j in range(RPT // 128):
          pltpu.sync_copy(rowsr_v.at[0].at[pl.ds(0, 128)],
                          acc_sh.at[pl.ds(row0 + j * 128, 128)])
        if with_cnt and k == 0:
          pltpu.sync_copy(z1d_v, cnt_sh.at[pl.ds(row0, RPT)])
        plsc.subcore_barrier()

        for p in range(DEPTH):
          pltpu.async_copy(tables[k].at[src_v.at[p]],
                           rowsr_v.at[p], sem_g)

        def step(g, carry, k=k):
          @pl.when(g >= DEPTH)
          def _drain():
            pltpu.make_async_copy(
                rowsr_v.at[lax.rem(g - DEPTH, NSLOT)],
                acc_sh.at[dst_v.at[g - DEPTH]], sem_s).wait()

          @pl.when(g + DEPTH < NG)
          def _prefetch():
            pltpu.async_copy(tables[k].at[src_v.at[g + DEPTH]],
                             rowsr_v.at[lax.rem(g + DEPTH, NSLOT)], sem_g)

          pltpu.make_async_copy(tables[k].at[src_v.at[g]],
                                rowsr_v.at[lax.rem(g, NSLOT)], sem_g).wait()
          pltpu.async_copy(rowsr_v.at[lax.rem(g, NSLOT)],
                           acc_sh.at[dst_v.at[g]], sem_s,
                           add=True)
          if with_cnt and k == 0:
            pltpu.async_copy(ones_v, cnt_sh.at[dst_v.at[g]],
                             sem_c, add=True)
          return carry

        lax.fori_loop(0, NG, step, 0)
        for g in range(max(0, NG - DEPTH), NG):
          pltpu.make_async_copy(rowsr_v.at[g % NSLOT],
                                acc_sh.at[dst_v.at[g]],
                                sem_s).wait()
        if with_cnt and k == 0:
          def drain_cnt(g, carry):
            pltpu.make_async_copy(ones_v, cnt_sh.at[dst_v.at[g]],
                                  sem_c).wait()
            return carry
          lax.fori_loop(0, NG, drain_cnt, 0)
        plsc.subcore_barrier()

        for j in range(RPT // 128):
          pltpu.sync_copy(acc_sh.at[pl.ds(row0 + j * 128, 128)],
                          rowsr_v.at[0].at[pl.ds(0, 128)])
          pltpu.sync_copy(rowsr_v.at[0].at[pl.ds(0, 128)],
                          outs[k].at[pl.ds(row0 + j * 128, 128)])
        if with_cnt and k == 0:
          pltpu.sync_copy(cnt_sh.at[pl.ds(row0, RPT)], z1d_v)
          pltpu.sync_copy(z1d_v, cnt_out.at[pl.ds(row0, RPT)])

    return None

  return pl.kernel(
      body, out_type=out_type, mesh=mesh, scratch_types=scratch,
      compiler_params=pltpu.CompilerParams(use_tc_tiling_on_sc=False))


_sc_agg_l1 = _make_sc_agg(NCH1, with_cnt=True)
_sc_agg_l2 = _make_sc_agg(NCH2, with_cnt=False)


def _tc_comb1(aggs, cnt, x, wr, wl, bl):
  def bodyfn(*refs):
    a_refs = refs[:NCH1]
    cnt_ref, x_ref, wr_ref, wl_ref, bl_ref = refs[NCH1:NCH1 + 5]
    h_refs = refs[NCH1 + 5:]
    inv = 1.0 / jnp.maximum(cnt_ref[...], 1.0)
    agg = jnp.concatenate([r[...] for r in a_refs], axis=1) * inv
    h = jnp.dot(agg, wl_ref[...], preferred_element_type=jnp.float32)
    h = h + jnp.dot(x_ref[...], wr_ref[...],
                    preferred_element_type=jnp.float32)
    h = jnp.maximum(h + bl_ref[...], 0.0)
    for k, hr in enumerate(h_refs):
      hr[...] = h[:, k * C:(k + 1) * C]

  return pl.pallas_call(
      bodyfn,
      grid=(G,),
      in_specs=[pl.BlockSpec((R, C), lambda i: (i, 0))] * NCH1 + [
          pl.BlockSpec((R, 1), lambda i: (i, 0)),
          pl.BlockSpec((R, IN_DIM), lambda i: (i, 0)),
          pl.BlockSpec((IN_DIM, HID_DIM), lambda i: (0, 0)),
          pl.BlockSpec((IN_DIM, HID_DIM), lambda i: (0, 0)),
          pl.BlockSpec((1, HID_DIM), lambda i: (0, 0)),
      ],
      out_specs=[pl.BlockSpec((R, C), lambda i: (i, 0))] * NCH2,
      out_shape=[jax.ShapeDtypeStruct((N_PAD, C), jnp.float32)] * NCH2,
  )(*aggs, cnt, x, wr, wl, bl)


def _tc_comb2(aggs, cnt, hs, wr, wl, bl):
  def bodyfn(*refs):
    a_refs = refs[:NCH2]
    h_refs = refs[NCH2:2 * NCH2]
    cnt_ref, wr_ref, wl_ref, bl_ref, out_ref = refs[2 * NCH2:2 * NCH2 + 5]
    inv = 1.0 / jnp.maximum(cnt_ref[...], 1.0)
    agg = jnp.concatenate([r[...] for r in a_refs], axis=1) * inv
    h = jnp.concatenate([r[...] for r in h_refs], axis=1)
    o = jnp.dot(agg, wl_ref[...], preferred_element_type=jnp.float32)
    o = o + jnp.dot(h, wr_ref[...], preferred_element_type=jnp.float32)
    out_ref[...] = o + bl_ref[...]

  return pl.pallas_call(
      bodyfn,
      grid=(G,),
      in_specs=[pl.BlockSpec((R, C), lambda i: (i, 0))] * (2 * NCH2) + [
          pl.BlockSpec((R, 1), lambda i: (i, 0)),
          pl.BlockSpec((HID_DIM, OUT_DIM), lambda i: (0, 0)),
          pl.BlockSpec((HID_DIM, OUT_DIM), lambda i: (0, 0)),
          pl.BlockSpec((1, OUT_DIM), lambda i: (0, 0)),
      ],
      out_specs=pl.BlockSpec((R, OUT_DIM), lambda i: (i, 0)),
      out_shape=jax.ShapeDtypeStruct((N, OUT_DIM), jnp.float32),
  )(*aggs, *hs, cnt, wr, wl, bl)


def kernel(x, edge_index, Wl1, bl1, Wr1, Wl2, bl2, Wr2):
  ei = edge_index.astype(jnp.int32)
  npad = E_PAD - E
  pad_iota = jnp.arange(npad, dtype=jnp.int32)
  src = jnp.concatenate(
      [ei[0], pad_iota % N]).reshape(NS, NG, KB)
  dst = jnp.concatenate(
      [ei[1], N + pad_iota % (N_PAD - N)]).reshape(NS, NG, KB)
  z2d = jnp.zeros((128, C), jnp.float32)
  ones1 = jnp.ones((KB,), jnp.float32)
  z1d = jnp.zeros((RPT,), jnp.float32)

  xc = [x[:, k * C:(k + 1) * C] for k in range(NCH1)]
  *a, cnt = _sc_agg_l1(src, dst, z2d, ones1, z1d, *xc)
  cnt2 = cnt.reshape(N_PAD, 1)

  h = _tc_comb1(a, cnt2, x, Wr1.T, Wl1.T, bl1.reshape(1, -1))

  g = _sc_agg_l2(src, dst, z2d, *h)

  return _tc_comb2(g, cnt2, h, Wr2.T, Wl2.T, bl2.reshape(1, -1))


# speedup vs baseline: 2.3050x; 1.0246x over previous
import jax
import jax.numpy as jnp
from jax import lax
from jax.experimental import pallas as pl
from jax.experimental.pallas import tpu as pltpu
from jax.experimental.pallas import tpu_sc as plsc

N = 10000
E = 160000
IN_DIM = 256
HID_DIM = 512
OUT_DIM = 256

NC = 2
NS = 16
B = 128
NB = 80
E_PAD = NS * NB * B
RPT = 640
N_PAD = NS * RPT
C = 64
K = 1
KB = K * B
NG = NB // K
NSLOT = 6
DEPTH = 3

R = 1024
G = 10

NCH1 = IN_DIM // C
NCH2 = HID_DIM // C


def _make_sc_agg(n_chunks, with_cnt):
  mesh = plsc.VectorSubcoreMesh(core_axis_name="c", subcore_axis_name="s")
  out_type = [jax.ShapeDtypeStruct((N_PAD, C), jnp.float32)
              for _ in range(n_chunks)]
  if with_cnt:
    out_type.append(jax.ShapeDtypeStruct((N_PAD,), jnp.float32))
  scratch = [
      pltpu.VMEM((NG, KB), jnp.int32),
      pltpu.VMEM((NG, KB), jnp.int32),
      pltpu.VMEM((NSLOT, KB, C), jnp.float32),
      pltpu.VMEM_SHARED((N_PAD, C), jnp.float32),
      pltpu.SemaphoreType.DMA,
      pltpu.SemaphoreType.DMA,
  ]
  if with_cnt:
    scratch += [
        pltpu.VMEM((KB,), jnp.float32),
        pltpu.VMEM((RPT,), jnp.float32),
        pltpu.VMEM_SHARED((N_PAD,), jnp.float32),
        pltpu.SemaphoreType.DMA,
    ]

  def body(*args):
    a = list(args)
    src_hbm, dst_hbm, z2d_hbm = a[:3]
    a = a[3:]
    if with_cnt:
      ones_hbm, z1d_hbm = a[:2]
      a = a[2:]
    tables = a[:n_chunks]
    a = a[n_chunks:]
    outs = a[:n_chunks]
    a = a[n_chunks:]
    if with_cnt:
      cnt_out = a[0]
      a = a[1:]
    src_v, dst_v, rowsr_v, acc_sh, sem_g, sem_s = a[:6]
    if with_cnt:
      ones_v, z1d_v, cnt_sh, sem_c = a[6:10]

    c = lax.axis_index("c")
    s = lax.axis_index("s")
    row0 = s * RPT

    pltpu.sync_copy(src_hbm.at[s], src_v)
    pltpu.sync_copy(dst_hbm.at[s], dst_v)
    if with_cnt:
      pltpu.sync_copy(ones_hbm, ones_v)
      pltpu.sync_copy(z1d_hbm, z1d_v)

    for k in range(n_chunks):
      @pl.when(c == (k % NC))
      def _chunk(k=k):
        pltpu.sync_copy(z2d_hbm.at[pl.ds(row0, RPT)],
                        acc_sh.at[pl.ds(row0, RPT)])
        if with_cnt and k == 0:
          pltpu.sync_copy(z1d_v, cnt_sh.at[pl.ds(row0, RPT)])
        plsc.subcore_barrier()

        for p in range(DEPTH):
          pltpu.async_copy(tables[k].at[src_v.at[p]],
                           rowsr_v.at[p], sem_g)

        def step(g, carry, k=k):
          @pl.when(g >= DEPTH)
          def _drain():
            pltpu.make_async_copy(
                rowsr_v.at[lax.rem(g - DEPTH, NSLOT)],
                acc_sh.at[dst_v.at[g - DEPTH]], sem_s).wait()

          @pl.when(g + DEPTH < NG)
          def _prefetch():
            pltpu.async_copy(tables[k].at[src_v.at[g + DEPTH]],
                             rowsr_v.at[lax.rem(g + DEPTH, NSLOT)], sem_g)

          pltpu.make_async_copy(tables[k].at[src_v.at[g]],
                                rowsr_v.at[lax.rem(g, NSLOT)], sem_g).wait()
          pltpu.async_copy(rowsr_v.at[lax.rem(g, NSLOT)],
                           acc_sh.at[dst_v.at[g]], sem_s,
                           add=True)
          if with_cnt and k == 0:
            pltpu.async_copy(ones_v, cnt_sh.at[dst_v.at[g]],
                             sem_c, add=True)
          return carry

        lax.fori_loop(0, NG, step, 0)
        for g in range(max(0, NG - DEPTH), NG):
          pltpu.make_async_copy(rowsr_v.at[g % NSLOT],
                                acc_sh.at[dst_v.at[g]],
                                sem_s).wait()
        if with_cnt and k == 0:
          def drain_cnt(g, carry):
            pltpu.make_async_copy(ones_v, cnt_sh.at[dst_v.at[g]],
                                  sem_c).wait()
            return carry
          lax.fori_loop(0, NG, drain_cnt, 0)
        plsc.subcore_barrier()

        pltpu.sync_copy(acc_sh.at[pl.ds(row0, RPT)],
                        outs[k].at[pl.ds(row0, RPT)])
        if with_cnt and k == 0:
          pltpu.sync_copy(cnt_sh.at[pl.ds(row0, RPT)],
                          cnt_out.at[pl.ds(row0, RPT)])

    return None

  return pl.kernel(
      body, out_type=out_type, mesh=mesh, scratch_types=scratch,
      compiler_params=pltpu.CompilerParams(use_tc_tiling_on_sc=False))


_sc_agg_l1 = _make_sc_agg(NCH1, with_cnt=True)
_sc_agg_l2 = _make_sc_agg(NCH2, with_cnt=False)


def _tc_comb1(aggs, cnt, x, wr, wl, bl):
  def bodyfn(*refs):
    a_refs = refs[:NCH1]
    cnt_ref, x_ref, wr_ref, wl_ref, bl_ref = refs[NCH1:NCH1 + 5]
    h_refs = refs[NCH1 + 5:]
    inv = 1.0 / jnp.maximum(cnt_ref[...], 1.0)
    agg = jnp.concatenate([r[...] for r in a_refs], axis=1) * inv
    h = jnp.dot(agg, wl_ref[...], preferred_element_type=jnp.float32)
    h = h + jnp.dot(x_ref[...], wr_ref[...],
                    preferred_element_type=jnp.float32)
    h = jnp.maximum(h + bl_ref[...], 0.0)
    for k, hr in enumerate(h_refs):
      hr[...] = h[:, k * C:(k + 1) * C]

  return pl.pallas_call(
      bodyfn,
      grid=(G,),
      in_specs=[pl.BlockSpec((R, C), lambda i: (i, 0))] * NCH1 + [
          pl.BlockSpec((R, 1), lambda i: (i, 0)),
          pl.BlockSpec((R, IN_DIM), lambda i: (i, 0)),
          pl.BlockSpec((IN_DIM, HID_DIM), lambda i: (0, 0)),
          pl.BlockSpec((IN_DIM, HID_DIM), lambda i: (0, 0)),
          pl.BlockSpec((1, HID_DIM), lambda i: (0, 0)),
      ],
      out_specs=[pl.BlockSpec((R, C), lambda i: (i, 0))] * NCH2,
      out_shape=[jax.ShapeDtypeStruct((N_PAD, C), jnp.float32)] * NCH2,
  )(*aggs, cnt, x, wr, wl, bl)


def _tc_comb2(aggs, cnt, hs, wr, wl, bl):
  def bodyfn(*refs):
    a_refs = refs[:NCH2]
    h_refs = refs[NCH2:2 * NCH2]
    cnt_ref, wr_ref, wl_ref, bl_ref, out_ref = refs[2 * NCH2:2 * NCH2 + 5]
    inv = 1.0 / jnp.maximum(cnt_ref[...], 1.0)
    agg = jnp.concatenate([r[...] for r in a_refs], axis=1) * inv
    h = jnp.concatenate([r[...] for r in h_refs], axis=1)
    o = jnp.dot(agg, wl_ref[...], preferred_element_type=jnp.float32)
    o = o + jnp.dot(h, wr_ref[...], preferred_element_type=jnp.float32)
    out_ref[...] = o + bl_ref[...]

  return pl.pallas_call(
      bodyfn,
      grid=(G,),
      in_specs=[pl.BlockSpec((R, C), lambda i: (i, 0))] * (2 * NCH2) + [
          pl.BlockSpec((R, 1), lambda i: (i, 0)),
          pl.BlockSpec((HID_DIM, OUT_DIM), lambda i: (0, 0)),
          pl.BlockSpec((HID_DIM, OUT_DIM), lambda i: (0, 0)),
          pl.BlockSpec((1, OUT_DIM), lambda i: (0, 0)),
      ],
      out_specs=pl.BlockSpec((R, OUT_DIM), lambda i: (i, 0)),
      out_shape=jax.ShapeDtypeStruct((N, OUT_DIM), jnp.float32),
  )(*aggs, *hs, cnt, wr, wl, bl)


def kernel(x, edge_index, Wl1, bl1, Wr1, Wl2, bl2, Wr2):
  ei = edge_index.astype(jnp.int32)
  npad = E_PAD - E
  pad_iota = jnp.arange(npad, dtype=jnp.int32)
  src = jnp.concatenate(
      [ei[0], pad_iota % N]).reshape(NS, NG, KB)
  dst = jnp.concatenate(
      [ei[1], N + pad_iota % (N_PAD - N)]).reshape(NS, NG, KB)
  z2d = jnp.zeros((N_PAD, C), jnp.float32)
  ones1 = jnp.ones((KB,), jnp.float32)
  z1d = jnp.zeros((RPT,), jnp.float32)

  xc = [x[:, k * C:(k + 1) * C] for k in range(NCH1)]
  *a, cnt = _sc_agg_l1(src, dst, z2d, ones1, z1d, *xc)
  cnt2 = cnt.reshape(N_PAD, 1)

  h = _tc_comb1(a, cnt2, x, Wr1.T, Wl1.T, bl1.reshape(1, -1))

  g = _sc_agg_l2(src, dst, z2d, *h)

  return _tc_comb2(g, cnt2, h, Wr2.T, Wl2.T, bl2.reshape(1, -1))
